# baseline (speedup 1.0000x reference)
import functools

import jax
import jax.numpy as jnp
from jax import lax
from jax.experimental import pallas as pl
from jax.experimental.pallas import tpu as pltpu
from jax.experimental.pallas import tpu_sc as plsc

K = 16
RB = 392
NSUB = 32
R_CH = 8
LANES = 16


def _normalize_body(x_ref, xn_ref, xnt_ref):
    v = x_ref[0]
    sq = jnp.sum(v * v, axis=0, keepdims=True)
    n = jnp.sqrt(sq)
    xn = v / jnp.maximum(n, 1e-12)
    xn_ref[0] = xn
    xnt_ref[0] = xn.T


def _dist_body(xnt_ref, xn_ref, nd_ref):
    lhs = xnt_ref[0]
    rhs = xn_ref[0]
    sqi = jnp.sum(lhs * lhs, axis=1, keepdims=True)
    sqj = jnp.sum(rhs * rhs, axis=0, keepdims=True)
    g = jax.lax.dot_general(lhs, rhs, (((1,), (0,)), ((), ())),
                            preferred_element_type=jnp.float32)
    d = (sqi + (-2.0 * g)) + sqj
    nd_ref[0] = -d


def _topk_sc_body(nd_hbm, out_hbm, buf, cv, ci, ob, *, n_points, rows_per):
    cid = lax.axis_index("c")
    sid = lax.axis_index("s")
    wid = sid * 2 + cid
    row0 = wid * rows_per
    n_chunks = n_points // LANES
    iota16 = lax.iota(jnp.int32, LANES)
    neg_inf = jnp.full((LANES,), -jnp.inf, jnp.float32)
    zeros_i = jnp.zeros((LANES,), jnp.int32)

    def group_body(g, _):
        pltpu.sync_copy(nd_hbm.at[pl.ds(row0 + g * R_CH, R_CH)], buf)

        def row_body(r, _):
            def p1(j, acc):
                return jnp.maximum(acc, buf[r, pl.ds(j * LANES, LANES)])
            acc = lax.fori_loop(0, n_chunks, p1, neg_inf)
            acc_sorted = lax.sort(acc)
            tv = jnp.full((LANES,), acc_sorted[0], jnp.float32)

            def p2(j, cnt):
                c = buf[r, pl.ds(j * LANES, LANES)]
                m = c >= tv
                idxv = iota16 + j * LANES
                plsc.store_compressed(cv.at[pl.ds(cnt, LANES)], c, mask=m)
                plsc.store_compressed(ci.at[pl.ds(cnt, LANES)], idxv, mask=m)
                return cnt + plsc.all_reduce_population_count(m)[0]
            cnt = lax.fori_loop(0, n_chunks, p2, jnp.int32(0))
            cv[pl.ds(cnt, LANES)] = neg_inf
            ci[pl.ds(cnt, LANES)] = zeros_i

            def p3(j, carry):
                sv, si = carry
                ck, cidx = plsc.sort_key_val(
                    cv[pl.ds(j * LANES, LANES)], ci[pl.ds(j * LANES, LANES)],
                    descending=True)
                sel = sv >= ck
                nv = jnp.where(sel, sv, ck)
                ni = jnp.where(sel, si, cidx)
                return tuple(plsc.sort_key_val(nv, ni, descending=False))
            nch = (cnt + 15) >> 4
            sv, si = lax.fori_loop(0, nch, p3, (neg_inf, zeros_i))
            ob[r, :] = lax.rev(si, (0,))
            return 0
        lax.fori_loop(0, R_CH, row_body, 0)
        pltpu.sync_copy(ob, out_hbm.at[pl.ds(row0 + g * R_CH, R_CH)])
        return 0
    lax.fori_loop(0, rows_per // R_CH, group_body, 0)


def kernel(x):
    B, C, H, W = x.shape
    N = H * W
    xf = x.reshape(B, C, N)

    xn, xnt = pl.pallas_call(
        _normalize_body,
        grid=(B,),
        in_specs=[pl.BlockSpec((1, C, N), lambda b: (b, 0, 0))],
        out_specs=[
            pl.BlockSpec((1, C, N), lambda b: (b, 0, 0)),
            pl.BlockSpec((1, N, C), lambda b: (b, 0, 0)),
        ],
        out_shape=[
            jax.ShapeDtypeStruct((B, C, N), jnp.float32),
            jax.ShapeDtypeStruct((B, N, C), jnp.float32),
        ],
    )(xf)

    nd = pl.pallas_call(
        _dist_body,
        grid=(B, N // RB),
        in_specs=[
            pl.BlockSpec((1, RB, C), lambda b, r: (b, r, 0)),
            pl.BlockSpec((1, C, N), lambda b, r: (b, 0, 0)),
        ],
        out_specs=pl.BlockSpec((1, RB, N), lambda b, r: (b, r, 0)),
        out_shape=jax.ShapeDtypeStruct((B, N, N), jnp.float32),
    )(xnt, xn)

    rows = B * N
    rows_per = rows // NSUB
    nd_flat = nd.reshape(rows, N)

    topk = pl.kernel(
        functools.partial(_topk_sc_body, n_points=N, rows_per=rows_per),
        out_type=jax.ShapeDtypeStruct((rows, K), jnp.int32),
        mesh=plsc.VectorSubcoreMesh(core_axis_name="c", subcore_axis_name="s",
                                    num_cores=2, num_subcores=16),
        compiler_params=pltpu.CompilerParams(needs_layout_passes=False),
        scratch_types=[
            pltpu.VMEM((R_CH, N), jnp.float32),
            pltpu.VMEM((N + LANES,), jnp.float32),
            pltpu.VMEM((N + LANES,), jnp.int32),
            pltpu.VMEM((R_CH, K), jnp.int32),
        ],
    )
    nn_idx = topk(nd_flat).reshape(B, N, K)

    center_idx = jnp.broadcast_to(
        jnp.arange(N, dtype=jnp.int32)[None, :, None], (B, N, K))
    return jnp.stack((nn_idx, center_idx), axis=0)


# speedup vs baseline: 1.6643x; 1.6643x over previous
import functools

import jax
import jax.numpy as jnp
from jax import lax
from jax.experimental import pallas as pl
from jax.experimental.pallas import tpu as pltpu
from jax.experimental.pallas import tpu_sc as plsc

K = 16
RB = 392
NSUB = 32
R_CH = 8
LANES = 16


def _normalize_body(x_ref, xn_ref, xnt_ref):
    v = x_ref[0]
    sq = jnp.sum(v * v, axis=0, keepdims=True)
    n = jnp.sqrt(sq)
    xn = v / jnp.maximum(n, 1e-12)
    xn_ref[0] = xn
    xnt_ref[0] = xn.T


def _dist_body(xnt_ref, xn_ref, nd_ref):
    lhs = xnt_ref[0]
    rhs = xn_ref[0]
    sqi = jnp.sum(lhs * lhs, axis=1, keepdims=True)
    sqj = jnp.sum(rhs * rhs, axis=0, keepdims=True)
    g = jax.lax.dot_general(lhs, rhs, (((1,), (0,)), ((), ())),
                            preferred_element_type=jnp.float32)
    d = (sqi + (-2.0 * g)) + sqj
    nd_ref[0] = -d


def _topk_sc_body(nd_hbm, out_hbm, buf, ci, ob, sems, *, n_points, rows_per):
    cid = lax.axis_index("c")
    sid = lax.axis_index("s")
    wid = sid * 2 + cid
    row0 = wid * rows_per
    n_groups = rows_per // R_CH
    n_chunks = n_points // LANES
    iota16 = lax.iota(jnp.int32, LANES)
    neg_inf = jnp.full((LANES,), -jnp.inf, jnp.float32)
    zeros_i = jnp.zeros((LANES,), jnp.int32)
    sixteen = jnp.full((LANES,), LANES, jnp.int32)

    pltpu.make_async_copy(
        nd_hbm.at[pl.ds(row0, R_CH)], buf.at[pl.ds(0, R_CH)],
        sems.at[0]).start()

    def group_body(g, _):
        cur = lax.rem(g, 2)
        base = cur * R_CH
        pltpu.make_async_copy(
            nd_hbm.at[pl.ds(row0 + g * R_CH, R_CH)],
            buf.at[pl.ds(base, R_CH)], sems.at[cur]).wait()

        @pl.when(g + 1 < n_groups)
        def _prefetch():
            pltpu.make_async_copy(
                nd_hbm.at[pl.ds(row0 + (g + 1) * R_CH, R_CH)],
                buf.at[pl.ds((1 - cur) * R_CH, R_CH)],
                sems.at[1 - cur]).start()

        def row_body(r, _):
            br = base + r

            def p1(j, acc):
                return jnp.maximum(acc, buf[br, pl.ds(j * LANES, LANES)])
            acc = lax.fori_loop(0, n_chunks, p1, neg_inf, unroll=14)
            acc_sorted = lax.sort(acc)
            tv = jnp.full((LANES,), acc_sorted[0], jnp.float32)

            def p2(j, carry):
                cnt, idxv = carry
                m = buf[br, pl.ds(j * LANES, LANES)] >= tv
                plsc.store_compressed(ci.at[pl.ds(cnt, LANES)], idxv, mask=m)
                return (cnt + plsc.all_reduce_population_count(m)[0],
                        idxv + sixteen)
            cnt, _ = lax.fori_loop(0, n_chunks, p2, (jnp.int32(0), iota16),
                                   unroll=7)
            ci[pl.ds(cnt, LANES)] = zeros_i
            br_splat = jnp.full((LANES,), br, jnp.int32)

            def p3(j, carry):
                sv, si = carry
                cidx_raw = ci[pl.ds(j * LANES, LANES)]
                valid = jnp.full((LANES,), j * LANES, jnp.int32) + iota16 < cnt
                cvals = jnp.where(
                    valid, plsc.load_gather(buf, [br_splat, cidx_raw]),
                    neg_inf)
                ck, cidx = plsc.sort_key_val(cvals, cidx_raw, descending=True)
                sel = sv >= ck
                nv = jnp.where(sel, sv, ck)
                ni = jnp.where(sel, si, cidx)
                return tuple(plsc.sort_key_val(nv, ni, descending=False))
            nch = (cnt + 15) >> 4
            sv, si = lax.fori_loop(0, nch, p3, (neg_inf, zeros_i))
            ob[g * R_CH + r, :] = lax.rev(si, (0,))
            return 0
        lax.fori_loop(0, R_CH, row_body, 0)
        return 0
    lax.fori_loop(0, n_groups, group_body, 0)
    pltpu.sync_copy(ob, out_hbm.at[pl.ds(row0, rows_per)])


def kernel(x):
    B, C, H, W = x.shape
    N = H * W
    xf = x.reshape(B, C, N)

    xn, xnt = pl.pallas_call(
        _normalize_body,
        grid=(B,),
        in_specs=[pl.BlockSpec((1, C, N), lambda b: (b, 0, 0))],
        out_specs=[
            pl.BlockSpec((1, C, N), lambda b: (b, 0, 0)),
            pl.BlockSpec((1, N, C), lambda b: (b, 0, 0)),
        ],
        out_shape=[
            jax.ShapeDtypeStruct((B, C, N), jnp.float32),
            jax.ShapeDtypeStruct((B, N, C), jnp.float32),
        ],
    )(xf)

    nd = pl.pallas_call(
        _dist_body,
        grid=(B, N // RB),
        in_specs=[
            pl.BlockSpec((1, RB, C), lambda b, r: (b, r, 0)),
            pl.BlockSpec((1, C, N), lambda b, r: (b, 0, 0)),
        ],
        out_specs=pl.BlockSpec((1, RB, N), lambda b, r: (b, r, 0)),
        out_shape=jax.ShapeDtypeStruct((B, N, N), jnp.float32),
    )(xnt, xn)

    rows = B * N
    rows_per = rows // NSUB
    nd_flat = nd.reshape(rows, N)

    topk = pl.kernel(
        functools.partial(_topk_sc_body, n_points=N, rows_per=rows_per),
        out_type=jax.ShapeDtypeStruct((rows, K), jnp.int32),
        mesh=plsc.VectorSubcoreMesh(core_axis_name="c", subcore_axis_name="s",
                                    num_cores=2, num_subcores=16),
        compiler_params=pltpu.CompilerParams(needs_layout_passes=False),
        scratch_types=[
            pltpu.VMEM((2 * R_CH, N), jnp.float32),
            pltpu.VMEM((N + LANES,), jnp.int32),
            pltpu.VMEM((rows_per, K), jnp.int32),
            pltpu.SemaphoreType.DMA((2,)),
        ],
    )
    nn_idx = topk(nd_flat).reshape(B, N, K)

    center_idx = jnp.broadcast_to(
        jnp.arange(N, dtype=jnp.int32)[None, :, None], (B, N, K))
    return jnp.stack((nn_idx, center_idx), axis=0)


# speedup vs baseline: 2.5299x; 1.5201x over previous
import functools

import jax
import jax.numpy as jnp
from jax import lax
from jax.experimental import pallas as pl
from jax.experimental.pallas import tpu as pltpu
from jax.experimental.pallas import tpu_sc as plsc

K = 16
RB = 392
NSUB = 32
R_CH = 8
LANES = 16


def _normalize_body(x_ref, xn_ref, xnt_ref):
    v = x_ref[0]
    sq = jnp.sum(v * v, axis=0, keepdims=True)
    n = jnp.sqrt(sq)
    xn = v / jnp.maximum(n, 1e-12)
    xn_ref[0] = xn
    xnt_ref[0] = xn.T


def _dist_body(xnt_ref, xn_ref, nd_ref):
    lhs = xnt_ref[0]
    rhs = xn_ref[0]
    sqi = jnp.sum(lhs * lhs, axis=1, keepdims=True)
    sqj = jnp.sum(rhs * rhs, axis=0, keepdims=True)
    g = jax.lax.dot_general(lhs, rhs, (((1,), (0,)), ((), ())),
                            preferred_element_type=jnp.float32)
    d = (sqi + (-2.0 * g)) + sqj
    nd_ref[0] = -d


def _topk_sc_body(nd_hbm, out_hbm, buf, cia, cib, ob, sems, *,
                  n_points, rows_per):
    cid = lax.axis_index("c")
    sid = lax.axis_index("s")
    wid = sid * 2 + cid
    row0 = wid * rows_per
    n_groups = rows_per // R_CH
    n_chunks = n_points // LANES
    iota16 = lax.iota(jnp.int32, LANES)
    neg_inf = jnp.full((LANES,), -jnp.inf, jnp.float32)
    zeros_i = jnp.zeros((LANES,), jnp.int32)
    sixteen = jnp.full((LANES,), LANES, jnp.int32)

    pltpu.make_async_copy(
        nd_hbm.at[pl.ds(row0, R_CH)], buf.at[pl.ds(0, R_CH)],
        sems.at[0]).start()

    def group_body(g, _):
        cur = lax.rem(g, 2)
        base = cur * R_CH
        pltpu.make_async_copy(
            nd_hbm.at[pl.ds(row0 + g * R_CH, R_CH)],
            buf.at[pl.ds(base, R_CH)], sems.at[cur]).wait()

        @pl.when(g + 1 < n_groups)
        def _prefetch():
            pltpu.make_async_copy(
                nd_hbm.at[pl.ds(row0 + (g + 1) * R_CH, R_CH)],
                buf.at[pl.ds((1 - cur) * R_CH, R_CH)],
                sems.at[1 - cur]).start()

        def pair_body(r2, _):
            bra = base + 2 * r2
            brb = bra + 1

            def p1(j, carry):
                aa, ab = carry
                return (jnp.maximum(aa, buf[bra, pl.ds(j * LANES, LANES)]),
                        jnp.maximum(ab, buf[brb, pl.ds(j * LANES, LANES)]))
            aa, ab = lax.fori_loop(0, n_chunks, p1, (neg_inf, neg_inf),
                                   unroll=14)
            tva = jnp.full((LANES,), lax.sort(aa)[0], jnp.float32)
            tvb = jnp.full((LANES,), lax.sort(ab)[0], jnp.float32)

            def p2(j, carry):
                cnta, cntb, idxv = carry
                ma = buf[bra, pl.ds(j * LANES, LANES)] >= tva
                mb = buf[brb, pl.ds(j * LANES, LANES)] >= tvb
                plsc.store_compressed(cia.at[pl.ds(cnta, LANES)], idxv,
                                      mask=ma)
                plsc.store_compressed(cib.at[pl.ds(cntb, LANES)], idxv,
                                      mask=mb)
                return (cnta + plsc.all_reduce_population_count(ma)[0],
                        cntb + plsc.all_reduce_population_count(mb)[0],
                        idxv + sixteen)
            cnta, cntb, _ = lax.fori_loop(
                0, n_chunks, p2, (jnp.int32(0), jnp.int32(0), iota16),
                unroll=7)
            cia[pl.ds(cnta, LANES)] = zeros_i
            cib[pl.ds(cntb, LANES)] = zeros_i

            def merge_rows(br, ci, cnt):
                br_splat = jnp.full((LANES,), br, jnp.int32)

                def p3(j, carry):
                    sv, si = carry
                    cidx_raw = ci[pl.ds(j * LANES, LANES)]
                    valid = (jnp.full((LANES,), j * LANES, jnp.int32)
                             + iota16 < cnt)
                    cvals = jnp.where(
                        valid, plsc.load_gather(buf, [br_splat, cidx_raw]),
                        neg_inf)
                    ck, cidx = plsc.sort_key_val(cvals, cidx_raw,
                                                 descending=True)
                    sel = sv >= ck
                    nv = jnp.where(sel, sv, ck)
                    ni = jnp.where(sel, si, cidx)
                    return tuple(plsc.sort_key_val(nv, ni, descending=False))
                nch = (cnt + 15) >> 4
                sv, si = lax.fori_loop(0, nch, p3, (neg_inf, zeros_i))
                return lax.rev(si, (0,))
            ob[g * R_CH + 2 * r2, :] = merge_rows(bra, cia, cnta)
            ob[g * R_CH + 2 * r2 + 1, :] = merge_rows(brb, cib, cntb)
            return 0
        lax.fori_loop(0, R_CH // 2, pair_body, 0)
        return 0
    lax.fori_loop(0, n_groups, group_body, 0)
    pltpu.sync_copy(ob, out_hbm.at[pl.ds(row0, rows_per)])


def kernel(x):
    B, C, H, W = x.shape
    N = H * W
    xf = x.reshape(B, C, N)

    xn, xnt = pl.pallas_call(
        _normalize_body,
        grid=(B,),
        in_specs=[pl.BlockSpec((1, C, N), lambda b: (b, 0, 0))],
        out_specs=[
            pl.BlockSpec((1, C, N), lambda b: (b, 0, 0)),
            pl.BlockSpec((1, N, C), lambda b: (b, 0, 0)),
        ],
        out_shape=[
            jax.ShapeDtypeStruct((B, C, N), jnp.float32),
            jax.ShapeDtypeStruct((B, N, C), jnp.float32),
        ],
    )(xf)

    nd = pl.pallas_call(
        _dist_body,
        grid=(B, N // RB),
        in_specs=[
            pl.BlockSpec((1, RB, C), lambda b, r: (b, r, 0)),
            pl.BlockSpec((1, C, N), lambda b, r: (b, 0, 0)),
        ],
        out_specs=pl.BlockSpec((1, RB, N), lambda b, r: (b, r, 0)),
        out_shape=jax.ShapeDtypeStruct((B, N, N), jnp.float32),
    )(xnt, xn)

    rows = B * N
    rows_per = rows // NSUB
    nd_flat = nd.reshape(rows, N)

    topk = pl.kernel(
        functools.partial(_topk_sc_body, n_points=N, rows_per=rows_per),
        out_type=jax.ShapeDtypeStruct((rows, K), jnp.int32),
        mesh=plsc.VectorSubcoreMesh(core_axis_name="c", subcore_axis_name="s",
                                    num_cores=2, num_subcores=16),
        compiler_params=pltpu.CompilerParams(needs_layout_passes=False),
        scratch_types=[
            pltpu.VMEM((2 * R_CH, N), jnp.float32),
            pltpu.VMEM((N + LANES,), jnp.int32),
            pltpu.VMEM((N + LANES,), jnp.int32),
            pltpu.VMEM((rows_per, K), jnp.int32),
            pltpu.SemaphoreType.DMA((2,)),
        ],
    )
    nn_idx = topk(nd_flat).reshape(B, N, K)

    center_idx = jnp.broadcast_to(
        jnp.arange(N, dtype=jnp.int32)[None, :, None], (B, N, K))
    return jnp.stack((nn_idx, center_idx), axis=0)
